# Initial kernel scaffold; baseline (speedup 1.0000x reference)
import jax
import jax.numpy as jnp
from jax.experimental import pallas as pl


def kernel(token_ids, p_emb):
    raise NotImplementedError("write your pallas kernel here")


# speedup vs baseline: 1.8589x; 1.8589x over previous
import functools

import jax
import jax.numpy as jnp
from jax import lax
from jax.experimental import pallas as pl
from jax.experimental.pallas import tpu as pltpu
from jax.experimental.pallas import tpu_sc as plsc

NC = 2
NS = 16
NW = NC * NS


IW = 128
K = 8


def _emb_call(n, d):
    n_per_w = n // NW
    c = K * IW
    n_groups = n_per_w // c
    mesh = plsc.VectorSubcoreMesh(
        core_axis_name="c", subcore_axis_name="s",
        num_cores=NC, num_subcores=NS)

    @functools.partial(
        pl.kernel,
        out_type=jax.ShapeDtypeStruct((n, d), jnp.float32),
        mesh=mesh,
        scratch_types=[
            pltpu.VMEM((n_per_w // IW, IW), jnp.int32),
            pltpu.VMEM((c, d), jnp.float32),
            pltpu.SemaphoreType.DMA,
        ],
        compiler_params=pltpu.CompilerParams(use_tc_tiling_on_sc=False),
    )
    def emb(ids_hbm, table_hbm, out_hbm, idx_v, rows_v, gsem):
        wid = lax.axis_index("s") * NC + lax.axis_index("c")
        base = wid * n_per_w
        pltpu.sync_copy(ids_hbm.at[wid], idx_v)

        def step(g, carry):
            copies = [
                pltpu.async_copy(
                    table_hbm.at[idx_v.at[g * K + j]],
                    rows_v.at[pl.ds(j * IW, IW)], gsem)
                for j in range(K)
            ]
            for cp in copies:
                cp.wait()
            pltpu.sync_copy(rows_v, out_hbm.at[pl.ds(base + g * c, c)])
            return carry

        lax.fori_loop(0, n_groups, step, 0)

    return emb


def kernel(token_ids, p_emb):
    b, h = token_ids.shape
    v, d = p_emb.shape
    n = b * h
    flat_ids = token_ids.reshape(NW, (n // NW) // IW, IW).astype(jnp.int32)
    out = _emb_call(n, d)(flat_ids, p_emb)
    return out.reshape(b, h, d)


# speedup vs baseline: 1.8743x; 1.0083x over previous
import functools

import jax
import jax.numpy as jnp
from jax import lax
from jax.experimental import pallas as pl
from jax.experimental.pallas import tpu as pltpu
from jax.experimental.pallas import tpu_sc as plsc

NC = 2
NS = 16
NW = NC * NS


IW = 128
K = 4


def _emb_call(n, d):
    n_per_w = n // NW
    c = K * IW
    n_groups = n_per_w // c
    assert n_groups % 2 == 0
    mesh = plsc.VectorSubcoreMesh(
        core_axis_name="c", subcore_axis_name="s",
        num_cores=NC, num_subcores=NS)

    @functools.partial(
        pl.kernel,
        out_type=jax.ShapeDtypeStruct((n, d), jnp.float32),
        mesh=mesh,
        scratch_types=[
            pltpu.VMEM((n_per_w // IW, IW), jnp.int32),
            pltpu.VMEM((2, c, d), jnp.float32),
            pltpu.SemaphoreType.DMA,
            pltpu.SemaphoreType.DMA,
        ],
        compiler_params=pltpu.CompilerParams(use_tc_tiling_on_sc=False),
    )
    def emb(ids_hbm, table_hbm, out_hbm, idx_v, rows_v, sem0, sem1):
        wid = lax.axis_index("s") * NC + lax.axis_index("c")
        base = wid * n_per_w
        pltpu.sync_copy(ids_hbm.at[wid], idx_v)

        bufs = (rows_v.at[0], rows_v.at[1])
        sems = (sem0, sem1)

        def fire(g, b):
            for j in range(K):
                pltpu.async_copy(
                    table_hbm.at[idx_v.at[g * K + j]],
                    bufs[b].at[pl.ds(j * IW, IW)], sems[b])

        def drain(g, b):
            for j in range(K):
                pltpu.make_async_copy(
                    table_hbm.at[idx_v.at[g * K + j]],
                    bufs[b].at[pl.ds(j * IW, IW)], sems[b]).wait()
            pltpu.sync_copy(bufs[b], out_hbm.at[pl.ds(base + g * c, c)])

        fire(0, 0)

        def step(i, carry):
            g = 2 * i
            fire(g + 1, 1)
            drain(g, 0)
            fire(g + 2, 0)
            drain(g + 1, 1)
            return carry

        lax.fori_loop(0, n_groups // 2 - 1, step, 0)
        g = n_groups - 2
        fire(g + 1, 1)
        drain(g, 0)
        drain(g + 1, 1)

    return emb


def kernel(token_ids, p_emb):
    b, h = token_ids.shape
    v, d = p_emb.shape
    n = b * h
    flat_ids = token_ids.reshape(NW, (n // NW) // IW, IW).astype(jnp.int32)
    out = _emb_call(n, d)(flat_ids, p_emb)
    return out.reshape(b, h, d)
